# baseline (speedup 1.0000x reference)
import jax
import jax.numpy as jnp
from jax.experimental import pallas as pl
from jax.experimental.pallas import tpu as pltpu

_E = 8
_D = 768
_H = 4 * _D
_T = 2048
_TM = 256
_TMAX = _T // _TM
_CS = 512


def _router(x, wg_ref, bg_ref, p0c, p1c, p0r, p1r, g0s, g1s, off_s, cnt_s):
    logits = jnp.dot(x, wg_ref[...], preferred_element_type=jnp.float32)
    logits = logits + bg_ref[...]
    col = jax.lax.broadcasted_iota(jnp.int32, (_T, _E), 1)

    m1 = jnp.max(logits, axis=1, keepdims=True)
    idx1 = jnp.min(jnp.where(logits == m1, col, _E), axis=1, keepdims=True)
    oh1 = col == idx1
    masked = jnp.where(oh1, -jnp.inf, logits)
    m2 = jnp.max(masked, axis=1, keepdims=True)
    idx2 = jnp.min(jnp.where(masked == m2, col, _E), axis=1, keepdims=True)
    oh2 = col == idx2

    e21 = jnp.exp(m2 - m1)
    g0s[...] = 1.0 / (1.0 + e21)
    g1s[...] = e21 / (1.0 + e21)

    o1 = oh1.astype(jnp.float32)
    o2 = oh2.astype(jnp.float32)

    row = jax.lax.broadcasted_iota(jnp.int32, (_CS, _CS), 0)
    colr = jax.lax.broadcasted_iota(jnp.int32, (_CS, _CS), 1)
    stl = (colr < row).astype(jnp.float32)
    run = jnp.zeros((1, _E), jnp.float32)
    ranks = []
    for onehot in (o1, o2):
        rblocks = []
        for b in range(_T // _CS):
            ob = jax.lax.slice(onehot, (b * _CS, 0), ((b + 1) * _CS, _E))
            rblocks.append(
                jnp.dot(stl, ob, preferred_element_type=jnp.float32) + run)
            run = run + jnp.sum(ob, axis=0, keepdims=True)
        ranks.append(jnp.concatenate(rblocks, axis=0))
    rank1, rank2 = ranks
    counts = run

    pc = jnp.ceil(counts / _TM) * _TM
    er = jax.lax.broadcasted_iota(jnp.int32, (_E, _E), 0)
    ec = jax.lax.broadcasted_iota(jnp.int32, (_E, _E), 1)
    excl = (er < ec).astype(jnp.float32)
    poff = jnp.dot(pc, excl, preferred_element_type=jnp.float32)

    pos0 = jnp.sum((rank1 + poff) * o1, axis=1, keepdims=True)
    pos1 = jnp.sum((rank2 + poff) * o2, axis=1, keepdims=True)
    pos0 = pos0.astype(jnp.int32)
    pos1 = pos1.astype(jnp.int32)
    p0c[...] = pos0
    p1c[...] = pos1
    p0r[...] = jnp.transpose(pos0, (1, 0))
    p1r[...] = jnp.transpose(pos1, (1, 0))

    poff_i = poff.astype(jnp.int32)
    pc_i = pc.astype(jnp.int32)
    for k in range(_E):
        off_s[k] = poff_i[0, k]
        cnt_s[k] = pc_i[0, k]


def _moe_body(x_ref, wg_ref, bg_ref, w1_ref, b1_ref, w2_ref, b2_ref, o_ref,
              p0c, p1c, p0r, p1r, g0s, g1s, off_s, cnt_s):
    e = pl.program_id(0)
    t = pl.program_id(1)

    @pl.when(jnp.logical_and(e == 0, t == 0))
    def _():
        _router(x_ref[...], wg_ref, bg_ref,
                p0c, p1c, p0r, p1r, g0s, g1s, off_s, cnt_s)
        o_ref[...] = jnp.zeros((_T, _D), jnp.float32)

    @pl.when(t * _TM < cnt_s[e])
    def _():
        r0 = off_s[e] + t * _TM
        rid = jax.lax.broadcasted_iota(jnp.int32, (_TM, _T), 0) + r0
        sel = jnp.logical_or(p0r[...] == rid, p1r[...] == rid)
        xs = jnp.dot(sel.astype(jnp.float32), x_ref[...],
                     preferred_element_type=jnp.float32)
        h = jnp.dot(xs, w1_ref[0],
                    preferred_element_type=jnp.float32) + b1_ref[0]
        h = jnp.maximum(h, 0.0)
        y = jnp.dot(h, w2_ref[0],
                    preferred_element_type=jnp.float32) + b2_ref[0]
        ridr = jax.lax.broadcasted_iota(jnp.int32, (_T, _TM), 1) + r0
        sgt = (jnp.where(p0c[...] == ridr, g0s[...], 0.0)
               + jnp.where(p1c[...] == ridr, g1s[...], 0.0))
        o_ref[...] += jnp.dot(sgt, y, preferred_element_type=jnp.float32)


def kernel(x, Wg, bg, W1, b1, W2, b2):
    b, s, d = x.shape
    x2d = x.reshape(_T, _D)
    bg2d = bg.reshape(1, _E)
    b13 = b1.reshape(_E, 1, _H)
    b23 = b2.reshape(_E, 1, _D)

    out = pl.pallas_call(
        _moe_body,
        grid=(_E, _TMAX),
        in_specs=[
            pl.BlockSpec((_T, _D), lambda e, t: (0, 0)),
            pl.BlockSpec((_D, _E), lambda e, t: (0, 0)),
            pl.BlockSpec((1, _E), lambda e, t: (0, 0)),
            pl.BlockSpec((1, _D, _H), lambda e, t: (e, 0, 0)),
            pl.BlockSpec((1, 1, _H), lambda e, t: (e, 0, 0)),
            pl.BlockSpec((1, _H, _D), lambda e, t: (e, 0, 0)),
            pl.BlockSpec((1, 1, _D), lambda e, t: (e, 0, 0)),
        ],
        out_specs=pl.BlockSpec((_T, _D), lambda e, t: (0, 0)),
        out_shape=jax.ShapeDtypeStruct((_T, _D), jnp.float32),
        scratch_shapes=[
            pltpu.VMEM((_T, 1), jnp.int32),
            pltpu.VMEM((_T, 1), jnp.int32),
            pltpu.VMEM((1, _T), jnp.int32),
            pltpu.VMEM((1, _T), jnp.int32),
            pltpu.VMEM((_T, 1), jnp.float32),
            pltpu.VMEM((_T, 1), jnp.float32),
            pltpu.SMEM((_E,), jnp.int32),
            pltpu.SMEM((_E,), jnp.int32),
        ],
        compiler_params=pltpu.CompilerParams(vmem_limit_bytes=64 * 2**20),
    )(x2d, Wg, bg2d, W1, b13, W2, b23)
    return out.reshape(b, s, d)


# speedup vs baseline: 1.1386x; 1.1386x over previous
import jax
import jax.numpy as jnp
from jax.experimental import pallas as pl
from jax.experimental.pallas import tpu as pltpu

_E = 8
_D = 768
_H = 4 * _D
_T = 2048
_TM = 256
_NT = 24
_NTP = 32
_ROWS = _NT * _TM
_CS = 512


def _router_body(x_ref, wg_ref, bg_ref,
                 pos0_ref, pos1_ref, g0_ref, g1_ref, te_ref,
                 rmap_ref, nu_ref):
    x = x_ref[...]
    logits = jnp.dot(x, wg_ref[...], preferred_element_type=jnp.float32)
    logits = logits + bg_ref[...]
    col = jax.lax.broadcasted_iota(jnp.int32, (_T, _E), 1)

    m1 = jnp.max(logits, axis=1, keepdims=True)
    idx1 = jnp.min(jnp.where(logits == m1, col, _E), axis=1, keepdims=True)
    oh1 = col == idx1
    masked = jnp.where(oh1, -jnp.inf, logits)
    m2 = jnp.max(masked, axis=1, keepdims=True)
    idx2 = jnp.min(jnp.where(masked == m2, col, _E), axis=1, keepdims=True)
    oh2 = col == idx2

    e21 = jnp.exp(m2 - m1)
    g0_ref[...] = 1.0 / (1.0 + e21)
    g1_ref[...] = e21 / (1.0 + e21)

    o1 = oh1.astype(jnp.float32)
    o2 = oh2.astype(jnp.float32)

    row = jax.lax.broadcasted_iota(jnp.int32, (_CS, _CS), 0)
    colr = jax.lax.broadcasted_iota(jnp.int32, (_CS, _CS), 1)
    stl = (colr < row).astype(jnp.float32)
    run = jnp.zeros((1, _E), jnp.float32)
    ranks = []
    for onehot in (o1, o2):
        rblocks = []
        for b in range(_T // _CS):
            ob = jax.lax.slice(onehot, (b * _CS, 0), ((b + 1) * _CS, _E))
            rblocks.append(
                jnp.dot(stl, ob, preferred_element_type=jnp.float32) + run)
            run = run + jnp.sum(ob, axis=0, keepdims=True)
        ranks.append(jnp.concatenate(rblocks, axis=0))
    rank1, rank2 = ranks
    counts = run

    pc = jnp.ceil(counts / _TM) * _TM
    er = jax.lax.broadcasted_iota(jnp.int32, (_E, _E), 0)
    ec = jax.lax.broadcasted_iota(jnp.int32, (_E, _E), 1)
    excl = (er < ec).astype(jnp.float32)
    poff = jnp.dot(pc, excl, preferred_element_type=jnp.float32)

    pos0 = jnp.sum((rank1 + poff) * o1, axis=1, keepdims=True)
    pos1 = jnp.sum((rank2 + poff) * o2, axis=1, keepdims=True)
    pos0_ref[...] = pos0.astype(jnp.int32)
    pos1_ref[...] = pos1.astype(jnp.int32)

    tend = (poff + pc) / _TM
    eye = (er == ec).astype(jnp.float32)
    tend_col = jnp.sum(jnp.broadcast_to(tend, (_E, _E)) * eye,
                       axis=1, keepdims=True)
    tid = jax.lax.broadcasted_iota(jnp.int32, (_E, _NTP), 1).astype(jnp.float32)
    ind = (tend_col <= tid).astype(jnp.int32)
    te = jnp.sum(ind, axis=0, keepdims=True)
    erow = jax.lax.broadcasted_iota(jnp.int32, (1, _E), 1)
    last_e = jnp.max(jnp.where(counts > 0, erow, 0), axis=1, keepdims=True)
    te_ref[...] = jnp.minimum(te, last_e)

    nu = (jnp.sum(pc, axis=1, keepdims=True) / _TM).astype(jnp.int32)
    nu_ref[...] = nu
    tid_i = jax.lax.broadcasted_iota(jnp.int32, (1, _NTP), 1)
    rmap_ref[...] = jnp.minimum(tid_i, nu - 1)


def _run_router(x2d, wg, bg2d):
    out_shapes = (
        jax.ShapeDtypeStruct((_T, 1), jnp.int32),
        jax.ShapeDtypeStruct((_T, 1), jnp.int32),
        jax.ShapeDtypeStruct((_T, 1), jnp.float32),
        jax.ShapeDtypeStruct((_T, 1), jnp.float32),
        jax.ShapeDtypeStruct((1, _NTP), jnp.int32),
        jax.ShapeDtypeStruct((1, _NTP), jnp.int32),
        jax.ShapeDtypeStruct((1, 1), jnp.int32),
    )
    return pl.pallas_call(
        _router_body,
        out_shape=out_shapes,
    )(x2d, wg, bg2d)


def _ffn_body(te_ref, rmap_ref, nu_ref,
              x_ref, p0r_ref, p1r_ref, p0c_ref, p1c_ref, g0_ref, g1_ref,
              w1_ref, b1_ref, w2_ref, b2_ref, o_ref):
    del te_ref
    i = pl.program_id(0)

    @pl.when(i < nu_ref[0])
    def _():
        r0 = rmap_ref[i] * _TM
        rid = jax.lax.broadcasted_iota(jnp.int32, (_TM, _T), 0) + r0
        sel = jnp.logical_or(p0r_ref[...] == rid, p1r_ref[...] == rid)
        xs = jnp.dot(sel.astype(jnp.float32), x_ref[...],
                     preferred_element_type=jnp.float32)
        h = jnp.dot(xs, w1_ref[0],
                    preferred_element_type=jnp.float32) + b1_ref[0]
        h = jnp.maximum(h, 0.0)
        y = jnp.dot(h, w2_ref[0],
                    preferred_element_type=jnp.float32) + b2_ref[0]

        ridr = jax.lax.broadcasted_iota(jnp.int32, (_T, _TM), 1) + r0
        sgt = (jnp.where(p0c_ref[...] == ridr, g0_ref[...], 0.0)
               + jnp.where(p1c_ref[...] == ridr, g1_ref[...], 0.0))
        contrib = jnp.dot(sgt, y, preferred_element_type=jnp.float32)

        @pl.when(i == 0)
        def _():
            o_ref[...] = contrib

        @pl.when(i > 0)
        def _():
            o_ref[...] += contrib


def _run_ffn(te, rmap, nu, x2d, p0r, p1r, p0c, p1c, g0, g1,
             w1, b13, w2, b23):
    grid_spec = pltpu.PrefetchScalarGridSpec(
        num_scalar_prefetch=3,
        grid=(_NT,),
        in_specs=[
            pl.BlockSpec((_T, _D), lambda i, te, rm, nu: (0, 0)),
            pl.BlockSpec((1, _T), lambda i, te, rm, nu: (0, 0)),
            pl.BlockSpec((1, _T), lambda i, te, rm, nu: (0, 0)),
            pl.BlockSpec((_T, 1), lambda i, te, rm, nu: (0, 0)),
            pl.BlockSpec((_T, 1), lambda i, te, rm, nu: (0, 0)),
            pl.BlockSpec((_T, 1), lambda i, te, rm, nu: (0, 0)),
            pl.BlockSpec((_T, 1), lambda i, te, rm, nu: (0, 0)),
            pl.BlockSpec((1, _D, _H), lambda i, te, rm, nu: (te[i], 0, 0)),
            pl.BlockSpec((1, 1, _H), lambda i, te, rm, nu: (te[i], 0, 0)),
            pl.BlockSpec((1, _H, _D), lambda i, te, rm, nu: (te[i], 0, 0)),
            pl.BlockSpec((1, 1, _D), lambda i, te, rm, nu: (te[i], 0, 0)),
        ],
        out_specs=pl.BlockSpec((_T, _D), lambda i, te, rm, nu: (0, 0)),
    )
    return pl.pallas_call(
        _ffn_body,
        grid_spec=grid_spec,
        out_shape=jax.ShapeDtypeStruct((_T, _D), jnp.float32),
        compiler_params=pltpu.CompilerParams(vmem_limit_bytes=64 * 2**20),
    )(te, rmap, nu, x2d, p0r, p1r, p0c, p1c, g0, g1, w1, b13, w2, b23)


def kernel(x, Wg, bg, W1, b1, W2, b2):
    b, s, d = x.shape
    x2d = x.reshape(_T, _D)
    bg2d = bg.reshape(1, _E)
    b13 = b1.reshape(_E, 1, _H)
    b23 = b2.reshape(_E, 1, _D)

    pos0, pos1, g0, g1, te, rmap, nu = _run_router(x2d, Wg, bg2d)

    out = _run_ffn(te.reshape(_NTP), rmap.reshape(_NTP), nu.reshape(1),
                   x2d, pos0.reshape(1, _T), pos1.reshape(1, _T),
                   pos0, pos1, g0, g1, W1, b13, W2, b23)
    return out.reshape(b, s, d)
